# baseline (speedup 1.0000x reference)
import jax
import jax.numpy as jnp
from jax.experimental import pallas as pl
from jax.experimental.pallas import tpu as pltpu

_LANES = 1024
_BLOCK_ROWS = 1024
_CH = 8
_UNROLL = 4


def _reduce_body(x_ref, out_ref, acc_o, acc_u):
    i = pl.program_id(0)

    def body(k, carry):
        ao, au = carry
        base = k * (_CH * _UNROLL)
        for j in range(_UNROLL):
            x = x_ref[pl.ds(base + j * _CH, _CH), :]
            t = x - 1.0
            o = jnp.maximum(t, 0.0)
            u = jnp.minimum(x, 0.0)
            ao = ao + o * o
            au = au + u * u
        return ao, au

    z = jnp.zeros((_CH, _LANES), jnp.float32)
    n_iter = _BLOCK_ROWS // (_CH * _UNROLL)
    ao, au = jax.lax.fori_loop(0, n_iter, body, (z, z))

    @pl.when(i == 0)
    def _init():
        acc_o[...] = jnp.zeros_like(acc_o)
        acc_u[...] = jnp.zeros_like(acc_u)

    acc_o[...] += ao
    acc_u[...] += au

    @pl.when(i == pl.num_programs(0) - 1)
    def _fini():
        s_o = jnp.sum(acc_o[...])
        s_u = jnp.sum(acc_u[...])
        out_ref[0, 0] = jnp.sqrt(s_o) + jnp.sqrt(s_u)


def kernel(tensor):
    n = tensor.size
    rows = n // _LANES
    x2d = tensor.reshape(rows, _LANES)
    grid = rows // _BLOCK_ROWS

    out = pl.pallas_call(
        _reduce_body,
        grid=(grid,),
        in_specs=[pl.BlockSpec((_BLOCK_ROWS, _LANES), lambda i: (i, 0))],
        out_specs=pl.BlockSpec(
            (1, 1), lambda i: (0, 0), memory_space=pltpu.SMEM
        ),
        out_shape=jax.ShapeDtypeStruct((1, 1), jnp.float32),
        scratch_shapes=[
            pltpu.VMEM((_CH, _LANES), jnp.float32),
            pltpu.VMEM((_CH, _LANES), jnp.float32),
        ],
        compiler_params=pltpu.CompilerParams(
            dimension_semantics=("arbitrary",),
        ),
    )(x2d)
    return out[0, 0]


# speedup vs baseline: 2.8917x; 2.8917x over previous
import jax
import jax.numpy as jnp
from jax.experimental import pallas as pl
from jax.experimental.pallas import tpu as pltpu

_LANES = 384
_BLOCK_ROWS = 2048
_CH = 8
_UNROLL = 4


def _reduce_body(x_ref, out_ref, acc_o, acc_u):
    i = pl.program_id(0)

    def body(k, carry):
        ao, au = carry
        base = k * (_CH * _UNROLL)
        for j in range(_UNROLL):
            x = x_ref[pl.ds(base + j * _CH, _CH), :]
            t = x - 1.0
            o = jnp.maximum(t, 0.0)
            u = jnp.minimum(x, 0.0)
            ao = ao + o * o
            au = au + u * u
        return ao, au

    z = jnp.zeros((_CH, _LANES), jnp.float32)
    n_iter = _BLOCK_ROWS // (_CH * _UNROLL)
    ao, au = jax.lax.fori_loop(0, n_iter, body, (z, z))

    @pl.when(i == 0)
    def _init():
        acc_o[...] = jnp.zeros_like(acc_o)
        acc_u[...] = jnp.zeros_like(acc_u)

    acc_o[...] += ao
    acc_u[...] += au

    @pl.when(i == pl.num_programs(0) - 1)
    def _fini():
        s_o = jnp.sum(acc_o[...])
        s_u = jnp.sum(acc_u[...])
        out_ref[0, 0] = jnp.sqrt(s_o) + jnp.sqrt(s_u)


def kernel(tensor):
    n = tensor.size
    rows = n // _LANES
    x2d = tensor.reshape(rows, _LANES)
    grid = rows // _BLOCK_ROWS

    out = pl.pallas_call(
        _reduce_body,
        grid=(grid,),
        in_specs=[pl.BlockSpec((_BLOCK_ROWS, _LANES), lambda i: (i, 0))],
        out_specs=pl.BlockSpec(
            (1, 1), lambda i: (0, 0), memory_space=pltpu.SMEM
        ),
        out_shape=jax.ShapeDtypeStruct((1, 1), jnp.float32),
        scratch_shapes=[
            pltpu.VMEM((_CH, _LANES), jnp.float32),
            pltpu.VMEM((_CH, _LANES), jnp.float32),
        ],
        compiler_params=pltpu.CompilerParams(
            dimension_semantics=("arbitrary",),
        ),
    )(x2d)
    return out[0, 0]


# speedup vs baseline: 4.1108x; 1.4216x over previous
import jax
import jax.numpy as jnp
from jax.experimental import pallas as pl
from jax.experimental.pallas import tpu as pltpu

_LANES = 384
_BLOCK_ROWS = 8192
_CH = 8
_UNROLL = 8


def _reduce_body(x_ref, out_ref, acc_o, acc_u):
    i = pl.program_id(0)

    def body(k, carry):
        ao, au = carry
        base = k * (_CH * _UNROLL)
        for j in range(_UNROLL):
            x = x_ref[pl.ds(base + j * _CH, _CH), :]
            t = x - 1.0
            o = jnp.maximum(t, 0.0)
            u = jnp.minimum(x, 0.0)
            ao = ao + o * o
            au = au + u * u
        return ao, au

    z = jnp.zeros((_CH, _LANES), jnp.float32)
    n_iter = _BLOCK_ROWS // (_CH * _UNROLL)
    ao, au = jax.lax.fori_loop(0, n_iter, body, (z, z))

    @pl.when(i == 0)
    def _init():
        acc_o[...] = jnp.zeros_like(acc_o)
        acc_u[...] = jnp.zeros_like(acc_u)

    acc_o[...] += ao
    acc_u[...] += au

    @pl.when(i == pl.num_programs(0) - 1)
    def _fini():
        s_o = jnp.sum(acc_o[...])
        s_u = jnp.sum(acc_u[...])
        out_ref[0, 0] = jnp.sqrt(s_o) + jnp.sqrt(s_u)


def kernel(tensor):
    n = tensor.size
    rows = n // _LANES
    x2d = tensor.reshape(rows, _LANES)
    grid = rows // _BLOCK_ROWS

    out = pl.pallas_call(
        _reduce_body,
        grid=(grid,),
        in_specs=[pl.BlockSpec((_BLOCK_ROWS, _LANES), lambda i: (i, 0))],
        out_specs=pl.BlockSpec(
            (1, 1), lambda i: (0, 0), memory_space=pltpu.SMEM
        ),
        out_shape=jax.ShapeDtypeStruct((1, 1), jnp.float32),
        scratch_shapes=[
            pltpu.VMEM((_CH, _LANES), jnp.float32),
            pltpu.VMEM((_CH, _LANES), jnp.float32),
        ],
        compiler_params=pltpu.CompilerParams(
            dimension_semantics=("arbitrary",),
        ),
    )(x2d)
    return out[0, 0]


# speedup vs baseline: 4.1303x; 1.0048x over previous
import jax
import jax.numpy as jnp
from jax.experimental import pallas as pl
from jax.experimental.pallas import tpu as pltpu

_LANES = 384
_BLOCK_ROWS = 16384
_CH = 8
_UNROLL = 8


def _reduce_body(x_ref, out_ref, acc_o, acc_u):
    i = pl.program_id(0)

    def body(k, carry):
        ao, au = carry
        base = k * (_CH * _UNROLL)
        for j in range(_UNROLL):
            x = x_ref[pl.ds(base + j * _CH, _CH), :]
            t = x - 1.0
            o = jnp.maximum(t, 0.0)
            u = jnp.minimum(x, 0.0)
            ao = ao + o * o
            au = au + u * u
        return ao, au

    z = jnp.zeros((_CH, _LANES), jnp.float32)
    n_iter = _BLOCK_ROWS // (_CH * _UNROLL)
    ao, au = jax.lax.fori_loop(0, n_iter, body, (z, z))

    @pl.when(i == 0)
    def _init():
        acc_o[...] = jnp.zeros_like(acc_o)
        acc_u[...] = jnp.zeros_like(acc_u)

    acc_o[...] += ao
    acc_u[...] += au

    @pl.when(i == pl.num_programs(0) - 1)
    def _fini():
        s_o = jnp.sum(acc_o[...])
        s_u = jnp.sum(acc_u[...])
        out_ref[0, 0] = jnp.sqrt(s_o) + jnp.sqrt(s_u)


def kernel(tensor):
    n = tensor.size
    rows = n // _LANES
    x2d = tensor.reshape(rows, _LANES)
    grid = rows // _BLOCK_ROWS

    out = pl.pallas_call(
        _reduce_body,
        grid=(grid,),
        in_specs=[pl.BlockSpec((_BLOCK_ROWS, _LANES), lambda i: (i, 0))],
        out_specs=pl.BlockSpec(
            (1, 1), lambda i: (0, 0), memory_space=pltpu.SMEM
        ),
        out_shape=jax.ShapeDtypeStruct((1, 1), jnp.float32),
        scratch_shapes=[
            pltpu.VMEM((_CH, _LANES), jnp.float32),
            pltpu.VMEM((_CH, _LANES), jnp.float32),
        ],
        compiler_params=pltpu.CompilerParams(
            dimension_semantics=("arbitrary",),
        ),
    )(x2d)
    return out[0, 0]
